# speedup vs baseline: 2.6559x; 2.6559x over previous
import jax
import jax.numpy as jnp
from jax.experimental import pallas as pl


_TB = 512


def _add_kernel(x_ref, pe_ref, o_ref):
    o_ref[...] = x_ref[...] + pe_ref[...]


def kernel(x, pos_table):
    b, t, d = x.shape
    grid = (t // _TB, b)
    return pl.pallas_call(
        _add_kernel,
        grid=grid,
        in_specs=[
            pl.BlockSpec((1, _TB, d), lambda i, j: (j, i, 0)),
            pl.BlockSpec((_TB, d), lambda i, j: (i, 0)),
        ],
        out_specs=pl.BlockSpec((1, _TB, d), lambda i, j: (j, i, 0)),
        out_shape=jax.ShapeDtypeStruct((b, t, d), x.dtype),
    )(x, pos_table)


# speedup vs baseline: 2.7634x; 1.0405x over previous
import jax
import jax.numpy as jnp
from jax.experimental import pallas as pl


_TB = 1024


def _add_kernel(x_ref, pe_ref, o_ref):
    o_ref[...] = x_ref[...] + pe_ref[...]


def kernel(x, pos_table):
    b, t, d = x.shape
    grid = (t // _TB, b)
    return pl.pallas_call(
        _add_kernel,
        grid=grid,
        in_specs=[
            pl.BlockSpec((1, _TB, d), lambda i, j: (j, i, 0)),
            pl.BlockSpec((_TB, d), lambda i, j: (i, 0)),
        ],
        out_specs=pl.BlockSpec((1, _TB, d), lambda i, j: (j, i, 0)),
        out_shape=jax.ShapeDtypeStruct((b, t, d), x.dtype),
    )(x, pos_table)
